# baseline (speedup 1.0000x reference)
import functools

import jax
import jax.numpy as jnp
from jax import lax
from jax.experimental import pallas as pl
from jax.experimental.pallas import tpu as pltpu
from jax.experimental.pallas import tpu_sc as plsc

_B0 = 16384
_B1 = 26
_B = _B0 * _B1
_D = 64
_V = 1000000
_NC = 2
_NS = 16
_NW = _NC * _NS
_CB = 128
_JBLK = _B0 // _CB
_UNITS = _B1 * _JBLK
_PER_W = _UNITS // _NW
_COLS_W = _B0 // _NW

_mesh = plsc.VectorSubcoreMesh(core_axis_name="c", subcore_axis_name="s")


@functools.partial(
    pl.kernel,
    mesh=_mesh,
    out_type=jax.ShapeDtypeStruct((_B,), jnp.int32),
    scratch_types=[
        pltpu.VMEM((_COLS_W,), jnp.int32),
        pltpu.VMEM((_COLS_W,), jnp.int32),
    ],
)
def _prep(idxT_hbm, out_hbm, v0, v1):
    wid = lax.axis_index("s") * _NC + lax.axis_index("c")
    col0 = wid * _COLS_W
    bufs = (v0, v1)
    for b1 in range(_B1):
        v = bufs[b1 % 2]
        pltpu.sync_copy(idxT_hbm.at[b1, pl.ds(col0, _COLS_W)], v)
        pltpu.sync_copy(v, out_hbm.at[pl.ds(b1 * _B0 + col0, _COLS_W)])


@functools.partial(
    pl.kernel,
    mesh=_mesh,
    out_type=jax.ShapeDtypeStruct((_B1, _D // 8, _JBLK, 8, _CB), jnp.float32),
    scratch_types=[
        pltpu.VMEM((_PER_W, _CB), jnp.int32),
        pltpu.VMEM((_CB, _D), jnp.float32),
        pltpu.VMEM((_CB, _D), jnp.float32),
        pltpu.VMEM((_D // 8, 8, _CB), jnp.float32),
        pltpu.VMEM((_D // 8, 8, _CB), jnp.float32),
        pltpu.SemaphoreType.DMA,
        pltpu.SemaphoreType.DMA,
        pltpu.SemaphoreType.DMA,
        pltpu.SemaphoreType.DMA,
    ],
    compiler_params=pltpu.CompilerParams(
        use_tc_tiling_on_sc=False, needs_layout_passes=False
    ),
)
def _gather(idx_hbm, table_hbm, out_hbm, idx_v, blk0, blk1, tr0, tr1,
            g0, g1, w0, w1):
    wid = lax.axis_index("s") * _NC + lax.axis_index("c")
    base_u = wid * _PER_W
    blk = (blk0, blk1)
    trs = (tr0, tr1)
    gsem = (g0, g1)
    wsem = (w0, w1)

    pltpu.sync_copy(idx_hbm.at[pl.ds(base_u, _PER_W), :], idx_v)

    iota = lax.iota(jnp.int32, 16)
    zero = iota * 0
    rows = [iota + 16 * k for k in range(_CB // 16)]

    def start_gather(u, b):
        pltpu.async_copy(table_hbm.at[idx_v.at[u]], blk[b], gsem[b])

    def wait_gather(u, b):
        pltpu.make_async_copy(
            table_hbm.at[idx_v.at[u]], blk[b], gsem[b]).wait()

    def start_write(u, b):
        g = base_u + u
        b1 = g // _JBLK
        j = g - b1 * _JBLK
        for i in range(_D // 8):
            pltpu.async_copy(trs[b].at[i], out_hbm.at[b1, i, j], wsem[b])

    def wait_write(u, b):
        g = base_u + u
        b1 = g // _JBLK
        j = g - b1 * _JBLK
        for i in range(_D // 8):
            pltpu.make_async_copy(
                trs[b].at[i], out_hbm.at[b1, i, j], wsem[b]).wait()

    def transpose(b):
        src = blk[b]
        dst = trs[b]
        for k in range(_CB // 16):
            rk = rows[k]
            for d in range(_D):
                v = plsc.load_gather(src, [rk, zero + d])
                dst[d // 8, d % 8, pl.ds(16 * k, 16)] = v

    def pair(i, carry):
        for b in range(2):
            u = i * 2 + b
            wait_gather(u, b)
            @pl.when(u + 1 < _PER_W)
            def _():
                start_gather(u + 1, 1 - b)
            @pl.when(u >= 2)
            def _():
                wait_write(u - 2, b)
            transpose(b)
            start_write(u, b)
        return carry

    start_gather(0, 0)
    lax.fori_loop(0, _PER_W // 2, pair, 0)
    wait_write(_PER_W - 2, 0)
    wait_write(_PER_W - 1, 1)


def kernel(indices, weight):
    idxT = indices.T.astype(jnp.int32)
    iflat = _prep(idxT).reshape(_UNITS, _CB)
    out5 = _gather(iflat, weight)
    return jnp.transpose(out5, (2, 4, 0, 1, 3)).reshape(_B0, _B1, _D)
